# baseline (speedup 1.0000x reference)
import functools

import jax
import jax.numpy as jnp
from jax import lax
from jax.experimental import pallas as pl
from jax.experimental.pallas import tpu as pltpu
from jax.experimental.pallas import tpu_sc as plsc

B = 16384
D = 32
B2 = 2 * B
V = 1000000
NW = 32
BPW = B2 // NW
CH = 128
NCH = BPW // CH
TW = 128
NG = 1954
TR = NG * TW

ATT = 64
BLK = 1024


def _tc_pack(tableT):
  def body(x_ref, o_ref):
    x = x_ref[...]
    o_ref[...] = jnp.concatenate(
        [jnp.swapaxes(x[:, a * TW:(a + 1) * TW], 0, 1) for a in range(4)],
        axis=1,
    )

  return pl.pallas_call(
      body,
      grid=(NG,),
      in_specs=[pl.BlockSpec((D, 4 * TW), lambda g: (0, g))],
      out_specs=pl.BlockSpec((TW, TW), lambda g: (g, 0)),
      out_shape=jax.ShapeDtypeStruct((TR, TW), jnp.float32),
  )(tableT)


def _sc_gather(table128, idx2d):
  mesh = plsc.VectorSubcoreMesh(core_axis_name="c", subcore_axis_name="s")

  @functools.partial(
      pl.kernel,
      mesh=mesh,
      out_type=jax.ShapeDtypeStruct((B2, TW), jnp.float32),
      scratch_types=[
          pltpu.VMEM((NCH, CH), jnp.int32),
          pltpu.VMEM((CH, TW), jnp.float32),
          pltpu.VMEM((CH, TW), jnp.float32),
          pltpu.SemaphoreType.DMA,
          pltpu.SemaphoreType.DMA,
          pltpu.SemaphoreType.DMA,
          pltpu.SemaphoreType.DMA,
      ],
  )
  def k(table_hbm, idx_hbm, out_hbm, idx_v, b0, b1, gs0, gs1, ws0, ws1):
    wid = lax.axis_index("s") * 2 + lax.axis_index("c")
    pltpu.sync_copy(idx_hbm.at[pl.ds(wid * NCH, NCH)], idx_v)
    bufs = (b0, b1)
    gsem = (gs0, gs1)
    wsem = (ws0, ws1)
    gc = [None, None]
    wc = [None, None]
    for j in range(NCH):
      p = j % 2
      if wc[p] is not None:
        wc[p].wait()
      gc[p] = pltpu.async_copy(table_hbm.at[idx_v.at[j]], bufs[p], gsem[p])
      if j >= 1:
        q = (j - 1) % 2
        gc[q].wait()
        wc[q] = pltpu.async_copy(
            bufs[q], out_hbm.at[pl.ds((wid * NCH + j - 1) * CH, CH)], wsem[q])
    q = (NCH - 1) % 2
    gc[q].wait()
    wc[q] = pltpu.async_copy(
        bufs[q], out_hbm.at[pl.ds((wid * NCH + NCH - 1) * CH, CH)], wsem[q])
    wc[0].wait()
    wc[1].wait()

  return k(table128, idx2d)


def _tc_scorer(em2, oh, w1c, b1r, w2, b2r):
  def body(em_ref, oh_ref, w1_ref, b1_ref, w2_ref, b2_ref, o_ref):
    em = em_ref[...]
    oh_ = oh_ref[...]
    parts = []
    for k in range(2):
      acc = em[:, k * TW:k * TW + D] * oh_[:, 4 * k:4 * k + 1]
      for q in range(1, 4):
        acc += em[:, k * TW + q * D:k * TW + (q + 1) * D] * \
            oh_[:, 4 * k + q:4 * k + q + 1]
      parts.append(acc)
    x = jnp.concatenate(parts, axis=1)
    h = jnp.tanh(
        jax.lax.dot_general(
            x, w1_ref[...], (((1,), (0,)), ((), ())),
            preferred_element_type=jnp.float32,
        )
        + b1_ref[...]
    )
    o_ref[...] = (
        jax.lax.dot_general(
            h, w2_ref[...], (((1,), (0,)), ((), ())),
            preferred_element_type=jnp.float32,
        )
        + b2_ref[...]
    )

  return pl.pallas_call(
      body,
      grid=(B // BLK,),
      in_specs=[
          pl.BlockSpec((BLK, 2 * TW), lambda i: (i, 0)),
          pl.BlockSpec((BLK, 8), lambda i: (i, 0)),
          pl.BlockSpec((2 * D, ATT), lambda i: (0, 0)),
          pl.BlockSpec((1, ATT), lambda i: (0, 0)),
          pl.BlockSpec((ATT, 1), lambda i: (0, 0)),
          pl.BlockSpec((1, 1), lambda i: (0, 0)),
      ],
      out_specs=pl.BlockSpec((BLK, 1), lambda i: (i, 0)),
      out_shape=jax.ShapeDtypeStruct((B, 1), jnp.float32),
  )(em2, oh, w1c, b1r, w2, b2r)


def kernel(inds, mask, table, W1, b1, W2, b2):
  table128 = _tc_pack(table.T)
  rows = ((inds >> 9) << 7) | (inds & 127)
  idx2d = rows.reshape(NW * NCH, CH)
  chunk = (inds >> 7) & 3
  oh = jax.nn.one_hot(chunk, 4, dtype=jnp.float32).reshape(B, 8)
  em2 = _sc_gather(table128, idx2d).reshape(B, 2 * TW)
  w1c = jnp.concatenate([W1, W1], axis=0) * 0.5
  return _tc_scorer(em2, oh, w1c, b1.reshape(1, ATT), W2, b2.reshape(1, 1))


# speedup vs baseline: 3.5977x; 3.5977x over previous
import functools

import jax
import jax.numpy as jnp
from jax import lax
from jax.experimental import pallas as pl
from jax.experimental.pallas import tpu as pltpu
from jax.experimental.pallas import tpu_sc as plsc

B = 16384
D = 32
B2 = 2 * B
V = 1000000
NW = 32
BPW = B2 // NW
CH = 128
NCH = BPW // CH
TW = 128
BLKC = 4096
NGP = (V + BLKC - 1) // BLKC
TR = NGP * (BLKC // 4)

ATT = 64
BLK = 1024


def _tc_pack(tableT, e4):
  def body(x_ref, e_ref, o_ref):
    x = x_ref[...]
    e = e_ref[...]
    acc = lax.dot_general(
        x[:, 0:1024], e[:, 0:128], (((0,), (0,)), ((), ())),
        preferred_element_type=jnp.float32)
    for v in range(1, 4):
      acc += lax.dot_general(
          x[:, 1024 * v:1024 * (v + 1)], e[:, 128 * v:128 * (v + 1)],
          (((0,), (0,)), ((), ())),
          preferred_element_type=jnp.float32)
    o_ref[...] = acc

  return pl.pallas_call(
      body,
      grid=(NGP,),
      in_specs=[
          pl.BlockSpec((D, BLKC), lambda g: (0, g)),
          pl.BlockSpec((D, 512), lambda g: (0, 0)),
      ],
      out_specs=pl.BlockSpec((BLKC // 4, TW), lambda g: (g, 0)),
      out_shape=jax.ShapeDtypeStruct((TR, TW), jnp.float32),
  )(tableT, e4)


def _sc_gather(table128, idx2d):
  mesh = plsc.VectorSubcoreMesh(core_axis_name="c", subcore_axis_name="s")

  @functools.partial(
      pl.kernel,
      mesh=mesh,
      out_type=jax.ShapeDtypeStruct((B2, TW), jnp.float32),
      scratch_types=[
          pltpu.VMEM((NCH, CH), jnp.int32),
          pltpu.VMEM((CH, TW), jnp.float32),
          pltpu.VMEM((CH, TW), jnp.float32),
          pltpu.SemaphoreType.DMA,
          pltpu.SemaphoreType.DMA,
          pltpu.SemaphoreType.DMA,
          pltpu.SemaphoreType.DMA,
      ],
  )
  def k(table_hbm, idx_hbm, out_hbm, idx_v, b0, b1, gs0, gs1, ws0, ws1):
    wid = lax.axis_index("s") * 2 + lax.axis_index("c")
    pltpu.sync_copy(idx_hbm.at[pl.ds(wid * NCH, NCH)], idx_v)
    bufs = (b0, b1)
    gsem = (gs0, gs1)
    wsem = (ws0, ws1)
    gc = [None, None]
    wc = [None, None]
    for j in range(NCH):
      p = j % 2
      if wc[p] is not None:
        wc[p].wait()
      gc[p] = pltpu.async_copy(table_hbm.at[idx_v.at[j]], bufs[p], gsem[p])
      if j >= 1:
        q = (j - 1) % 2
        gc[q].wait()
        wc[q] = pltpu.async_copy(
            bufs[q], out_hbm.at[pl.ds((wid * NCH + j - 1) * CH, CH)], wsem[q])
    q = (NCH - 1) % 2
    gc[q].wait()
    wc[q] = pltpu.async_copy(
        bufs[q], out_hbm.at[pl.ds((wid * NCH + NCH - 1) * CH, CH)], wsem[q])
    wc[0].wait()
    wc[1].wait()

  return k(table128, idx2d)


def _tc_scorer(em2, grp, w1rep, b1r, w2, b2r):
  def body(em_ref, g_ref, w1_ref, b1_ref, w2_ref, b2_ref, o_ref):
    em = em_ref[...]
    g0 = g_ref[:, 0:1]
    g1 = g_ref[:, 1:2]
    lane = lax.broadcasted_iota(jnp.int32, (BLK, 2 * TW), 1)
    want = jnp.where(lane < TW, g0, g1)
    x = jnp.where(((lane >> 5) & 3) == want, em, 0.0)
    h = jnp.tanh(
        lax.dot_general(
            x, w1_ref[...], (((1,), (0,)), ((), ())),
            preferred_element_type=jnp.float32,
        )
        + b1_ref[...]
    )
    o_ref[...] = (
        lax.dot_general(
            h, w2_ref[...], (((1,), (0,)), ((), ())),
            preferred_element_type=jnp.float32,
        )
        + b2_ref[...]
    )

  return pl.pallas_call(
      body,
      grid=(B // BLK,),
      in_specs=[
          pl.BlockSpec((BLK, 2 * TW), lambda i: (i, 0)),
          pl.BlockSpec((BLK, 2), lambda i: (i, 0)),
          pl.BlockSpec((2 * TW, ATT), lambda i: (0, 0)),
          pl.BlockSpec((1, ATT), lambda i: (0, 0)),
          pl.BlockSpec((ATT, 1), lambda i: (0, 0)),
          pl.BlockSpec((1, 1), lambda i: (0, 0)),
      ],
      out_specs=pl.BlockSpec((BLK, 1), lambda i: (i, 0)),
      out_shape=jax.ShapeDtypeStruct((B, 1), jnp.float32),
  )(em2, grp, w1rep, b1r, w2, b2r)


def kernel(inds, mask, table, W1, b1, W2, b2):
  eye = jnp.eye(D, dtype=jnp.float32)
  e4 = jnp.concatenate(
      [jnp.pad(eye, ((0, 0), (32 * v, 96 - 32 * v))) for v in range(4)],
      axis=1)
  table128 = _tc_pack(table.T, e4)
  rows = ((inds >> 12) << 10) | (inds & 1023)
  idx2d = rows.reshape(NW * NCH, CH)
  grp = (inds >> 10) & 3
  em2 = _sc_gather(table128, idx2d).reshape(B, 2 * TW)
  w1rep = jnp.tile(W1, (8, 1)) * 0.5
  return _tc_scorer(em2, grp, w1rep, b1.reshape(1, ATT), W2,
                    b2.reshape(1, 1))


# speedup vs baseline: 3.9186x; 1.0892x over previous
import functools

import jax
import jax.numpy as jnp
from jax import lax
from jax.experimental import pallas as pl
from jax.experimental.pallas import tpu as pltpu
from jax.experimental.pallas import tpu_sc as plsc

B = 16384
D = 32
B2 = 2 * B
V = 1000000
NW = 32
BPW = B2 // NW
CH = 128
NCH = BPW // CH
TW = 128
BLKC = 4096
NGP = (V + BLKC - 1) // BLKC
TR = NGP * (BLKC // 4)

ATT = 64
BLK = 1024


def _tc_pack(tableT, e4):
  def body(x_ref, e_ref, o_ref):
    x = x_ref[...].astype(jnp.bfloat16)
    e = e_ref[...]
    acc = lax.dot_general(
        x[:, 0:1024], e[:, 0:128], (((0,), (0,)), ((), ())),
        preferred_element_type=jnp.float32)
    for v in range(1, 4):
      acc += lax.dot_general(
          x[:, 1024 * v:1024 * (v + 1)], e[:, 128 * v:128 * (v + 1)],
          (((0,), (0,)), ((), ())),
          preferred_element_type=jnp.float32)
    o_ref[...] = acc

  return pl.pallas_call(
      body,
      grid=(NGP,),
      in_specs=[
          pl.BlockSpec((D, BLKC), lambda g: (0, g)),
          pl.BlockSpec((D, 512), lambda g: (0, 0)),
      ],
      out_specs=pl.BlockSpec((BLKC // 4, TW), lambda g: (g, 0)),
      out_shape=jax.ShapeDtypeStruct((TR, TW), jnp.float32),
  )(tableT, e4)


def _sc_gather(table128, idx2d):
  mesh = plsc.VectorSubcoreMesh(core_axis_name="c", subcore_axis_name="s")

  @functools.partial(
      pl.kernel,
      mesh=mesh,
      out_type=jax.ShapeDtypeStruct((B2, TW), jnp.float32),
      scratch_types=[
          pltpu.VMEM((NCH, CH), jnp.int32),
          pltpu.VMEM((CH, TW), jnp.float32),
          pltpu.VMEM((CH, TW), jnp.float32),
          pltpu.SemaphoreType.DMA,
          pltpu.SemaphoreType.DMA,
          pltpu.SemaphoreType.DMA,
          pltpu.SemaphoreType.DMA,
      ],
  )
  def k(table_hbm, idx_hbm, out_hbm, idx_v, b0, b1, gs0, gs1, ws0, ws1):
    wid = lax.axis_index("s") * 2 + lax.axis_index("c")
    pltpu.sync_copy(idx_hbm.at[pl.ds(wid * NCH, NCH)], idx_v)
    bufs = (b0, b1)
    gsem = (gs0, gs1)
    wsem = (ws0, ws1)
    gc = [None, None]
    wc = [None, None]
    for j in range(NCH):
      p = j % 2
      if wc[p] is not None:
        wc[p].wait()
      gc[p] = pltpu.async_copy(table_hbm.at[idx_v.at[j]], bufs[p], gsem[p])
      if j >= 1:
        q = (j - 1) % 2
        gc[q].wait()
        wc[q] = pltpu.async_copy(
            bufs[q], out_hbm.at[pl.ds((wid * NCH + j - 1) * CH, CH)], wsem[q])
    q = (NCH - 1) % 2
    gc[q].wait()
    wc[q] = pltpu.async_copy(
        bufs[q], out_hbm.at[pl.ds((wid * NCH + NCH - 1) * CH, CH)], wsem[q])
    wc[0].wait()
    wc[1].wait()

  return k(table128, idx2d)


def _tc_scorer(em2, grp, w1rep, b1r, w2, b2r):
  def body(em_ref, g_ref, w1_ref, b1_ref, w2_ref, b2_ref, o_ref):
    em = em_ref[...]
    g0 = g_ref[:, 0:1]
    g1 = g_ref[:, 1:2]
    lane = lax.broadcasted_iota(jnp.int32, (BLK, 2 * TW), 1)
    want = jnp.where(lane < TW, g0, g1)
    x = jnp.where(((lane >> 5) & 3) == want, em, 0.0)
    h = jnp.tanh(
        lax.dot_general(
            x, w1_ref[...], (((1,), (0,)), ((), ())),
            preferred_element_type=jnp.float32,
        )
        + b1_ref[...]
    )
    o_ref[...] = (
        lax.dot_general(
            h, w2_ref[...], (((1,), (0,)), ((), ())),
            preferred_element_type=jnp.float32,
        )
        + b2_ref[...]
    )

  return pl.pallas_call(
      body,
      grid=(B // BLK,),
      in_specs=[
          pl.BlockSpec((BLK, 2 * TW), lambda i: (i, 0)),
          pl.BlockSpec((BLK, 2), lambda i: (i, 0)),
          pl.BlockSpec((2 * TW, ATT), lambda i: (0, 0)),
          pl.BlockSpec((1, ATT), lambda i: (0, 0)),
          pl.BlockSpec((ATT, 1), lambda i: (0, 0)),
          pl.BlockSpec((1, 1), lambda i: (0, 0)),
      ],
      out_specs=pl.BlockSpec((BLK, 1), lambda i: (i, 0)),
      out_shape=jax.ShapeDtypeStruct((B, 1), jnp.float32),
  )(em2, grp, w1rep, b1r, w2, b2r)


def kernel(inds, mask, table, W1, b1, W2, b2):
  eye = jnp.eye(D, dtype=jnp.float32)
  e4 = jnp.concatenate(
      [jnp.pad(eye, ((0, 0), (32 * v, 96 - 32 * v))) for v in range(4)],
      axis=1).astype(jnp.bfloat16)
  table128 = _tc_pack(table.T, e4)
  rows = ((inds >> 12) << 10) | (inds & 1023)
  idx2d = rows.reshape(NW * NCH, CH)
  grp = (inds >> 10) & 3
  em2 = _sc_gather(table128, idx2d).reshape(B, 2 * TW)
  w1rep = jnp.tile(W1, (8, 1)) * 0.5
  return _tc_scorer(em2, grp, w1rep, b1.reshape(1, ATT), W2,
                    b2.reshape(1, 1))


# speedup vs baseline: 5.7986x; 1.4798x over previous
import functools

import jax
import jax.numpy as jnp
from jax import lax
from jax.experimental import pallas as pl
from jax.experimental.pallas import tpu as pltpu
from jax.experimental.pallas import tpu_sc as plsc

B = 16384
D = 32
B2 = 2 * B
V = 1000000
NW = 32
BPW = B2 // NW
CH = 128
NCH = BPW // CH
TW = 128
BLKC = 16384
GW = BLKC // 4
SH = BLKC.bit_length() - 1
NGP = (V + BLKC - 1) // BLKC
TR = NGP * GW

ATT = 64
BLK = 1024


def _tc_pack(tableT, e4):
  def body(x_ref, e_ref, o_ref):
    x = x_ref[...].astype(jnp.bfloat16)
    e = e_ref[...]
    acc = lax.dot_general(
        x[:, 0:GW], e[:, 0:128], (((0,), (0,)), ((), ())),
        preferred_element_type=jnp.float32)
    for v in range(1, 4):
      acc += lax.dot_general(
          x[:, GW * v:GW * (v + 1)], e[:, 128 * v:128 * (v + 1)],
          (((0,), (0,)), ((), ())),
          preferred_element_type=jnp.float32)
    o_ref[...] = acc

  return pl.pallas_call(
      body,
      grid=(NGP,),
      in_specs=[
          pl.BlockSpec((D, BLKC), lambda g: (0, g)),
          pl.BlockSpec((D, 512), lambda g: (0, 0)),
      ],
      out_specs=pl.BlockSpec((GW, TW), lambda g: (g, 0)),
      out_shape=jax.ShapeDtypeStruct((TR, TW), jnp.float32),
  )(tableT, e4)


def _sc_gather(table128, idx2d):
  mesh = plsc.VectorSubcoreMesh(core_axis_name="c", subcore_axis_name="s")

  @functools.partial(
      pl.kernel,
      mesh=mesh,
      out_type=jax.ShapeDtypeStruct((B2, TW), jnp.float32),
      scratch_types=[
          pltpu.VMEM((NCH, CH), jnp.int32),
          pltpu.VMEM((CH, TW), jnp.float32),
          pltpu.VMEM((CH, TW), jnp.float32),
          pltpu.SemaphoreType.DMA,
          pltpu.SemaphoreType.DMA,
          pltpu.SemaphoreType.DMA,
          pltpu.SemaphoreType.DMA,
      ],
  )
  def k(table_hbm, idx_hbm, out_hbm, idx_v, b0, b1, gs0, gs1, ws0, ws1):
    wid = lax.axis_index("s") * 2 + lax.axis_index("c")
    pltpu.sync_copy(idx_hbm.at[pl.ds(wid * NCH, NCH)], idx_v)
    bufs = (b0, b1)
    gsem = (gs0, gs1)
    wsem = (ws0, ws1)
    gc = [None, None]
    wc = [None, None]
    for j in range(NCH):
      p = j % 2
      if wc[p] is not None:
        wc[p].wait()
      gc[p] = pltpu.async_copy(table_hbm.at[idx_v.at[j]], bufs[p], gsem[p])
      if j >= 1:
        q = (j - 1) % 2
        gc[q].wait()
        wc[q] = pltpu.async_copy(
            bufs[q], out_hbm.at[pl.ds((wid * NCH + j - 1) * CH, CH)], wsem[q])
    q = (NCH - 1) % 2
    gc[q].wait()
    wc[q] = pltpu.async_copy(
        bufs[q], out_hbm.at[pl.ds((wid * NCH + NCH - 1) * CH, CH)], wsem[q])
    wc[0].wait()
    wc[1].wait()

  return k(table128, idx2d)


def _tc_scorer(em2, grp, w1rep, b1r, w2, b2r):
  def body(em_ref, g_ref, w1_ref, b1_ref, w2_ref, b2_ref, o_ref):
    em = em_ref[...]
    g0 = g_ref[:, 0:1]
    g1 = g_ref[:, 1:2]
    lane = lax.broadcasted_iota(jnp.int32, (BLK, 2 * TW), 1)
    want = jnp.where(lane < TW, g0, g1)
    x = jnp.where(((lane >> 5) & 3) == want, em, 0.0)
    h = jnp.tanh(
        lax.dot_general(
            x, w1_ref[...], (((1,), (0,)), ((), ())),
            preferred_element_type=jnp.float32,
        )
        + b1_ref[...]
    )
    o_ref[...] = (
        lax.dot_general(
            h, w2_ref[...], (((1,), (0,)), ((), ())),
            preferred_element_type=jnp.float32,
        )
        + b2_ref[...]
    )

  return pl.pallas_call(
      body,
      grid=(B // BLK,),
      in_specs=[
          pl.BlockSpec((BLK, 2 * TW), lambda i: (i, 0)),
          pl.BlockSpec((BLK, 2), lambda i: (i, 0)),
          pl.BlockSpec((2 * TW, ATT), lambda i: (0, 0)),
          pl.BlockSpec((1, ATT), lambda i: (0, 0)),
          pl.BlockSpec((ATT, 1), lambda i: (0, 0)),
          pl.BlockSpec((1, 1), lambda i: (0, 0)),
      ],
      out_specs=pl.BlockSpec((BLK, 1), lambda i: (i, 0)),
      out_shape=jax.ShapeDtypeStruct((B, 1), jnp.float32),
  )(em2, grp, w1rep, b1r, w2, b2r)


def kernel(inds, mask, table, W1, b1, W2, b2):
  eye = jnp.eye(D, dtype=jnp.float32)
  e4 = jnp.concatenate(
      [jnp.pad(eye, ((0, 0), (32 * v, 96 - 32 * v))) for v in range(4)],
      axis=1).astype(jnp.bfloat16)
  table128 = _tc_pack(table.T, e4)
  rows = ((inds >> SH) << (SH - 2)) | (inds & (GW - 1))
  idx2d = rows.reshape(NW * NCH, CH)
  grp = (inds >> (SH - 2)) & 3
  em2 = _sc_gather(table128, idx2d).reshape(B, 2 * TW)
  w1rep = jnp.tile(W1, (8, 1)) * 0.5
  return _tc_scorer(em2, grp, w1rep, b1.reshape(1, ATT), W2,
                    b2.reshape(1, 1))


# speedup vs baseline: 6.3561x; 1.0961x over previous
import functools

import jax
import jax.numpy as jnp
from jax import lax
from jax.experimental import pallas as pl
from jax.experimental.pallas import tpu as pltpu
from jax.experimental.pallas import tpu_sc as plsc

B = 16384
D = 32
B2 = 2 * B
V = 1000000
NW = 32
BPW = B2 // NW
CH = 128
NCH = BPW // CH
TW = 128
BLKC = 32768
GW = BLKC // 4
SH = BLKC.bit_length() - 1
NGP = (V + BLKC - 1) // BLKC
TR = NGP * GW

ATT = 64
BLK = 1024


def _tc_pack(tableT, e4):
  def body(x_ref, e_ref, o_ref):
    x = x_ref[...].astype(jnp.bfloat16)
    e = e_ref[...]
    acc = lax.dot_general(
        x[:, 0:GW], e[:, 0:128], (((0,), (0,)), ((), ())),
        preferred_element_type=jnp.float32)
    for v in range(1, 4):
      acc += lax.dot_general(
          x[:, GW * v:GW * (v + 1)], e[:, 128 * v:128 * (v + 1)],
          (((0,), (0,)), ((), ())),
          preferred_element_type=jnp.float32)
    o_ref[...] = acc

  return pl.pallas_call(
      body,
      grid=(NGP,),
      in_specs=[
          pl.BlockSpec((D, BLKC), lambda g: (0, g)),
          pl.BlockSpec((D, 512), lambda g: (0, 0)),
      ],
      out_specs=pl.BlockSpec((GW, TW), lambda g: (g, 0)),
      out_shape=jax.ShapeDtypeStruct((TR, TW), jnp.float32),
  )(tableT, e4)


def _sc_gather(table128, idx2d):
  mesh = plsc.VectorSubcoreMesh(core_axis_name="c", subcore_axis_name="s")

  @functools.partial(
      pl.kernel,
      mesh=mesh,
      out_type=jax.ShapeDtypeStruct((B2, TW), jnp.float32),
      scratch_types=[
          pltpu.VMEM((NCH, CH), jnp.int32),
          pltpu.VMEM((CH, TW), jnp.float32),
          pltpu.VMEM((CH, TW), jnp.float32),
          pltpu.SemaphoreType.DMA,
          pltpu.SemaphoreType.DMA,
          pltpu.SemaphoreType.DMA,
          pltpu.SemaphoreType.DMA,
      ],
  )
  def k(table_hbm, idx_hbm, out_hbm, idx_v, b0, b1, gs0, gs1, ws0, ws1):
    wid = lax.axis_index("s") * 2 + lax.axis_index("c")
    pltpu.sync_copy(idx_hbm.at[pl.ds(wid * NCH, NCH)], idx_v)
    bufs = (b0, b1)
    gsem = (gs0, gs1)
    wsem = (ws0, ws1)
    gc = [None, None]
    wc = [None, None]
    for j in range(NCH):
      p = j % 2
      if wc[p] is not None:
        wc[p].wait()
      gc[p] = pltpu.async_copy(table_hbm.at[idx_v.at[j]], bufs[p], gsem[p])
      if j >= 1:
        q = (j - 1) % 2
        gc[q].wait()
        wc[q] = pltpu.async_copy(
            bufs[q], out_hbm.at[pl.ds((wid * NCH + j - 1) * CH, CH)], wsem[q])
    q = (NCH - 1) % 2
    gc[q].wait()
    wc[q] = pltpu.async_copy(
        bufs[q], out_hbm.at[pl.ds((wid * NCH + NCH - 1) * CH, CH)], wsem[q])
    wc[0].wait()
    wc[1].wait()

  return k(table128, idx2d)


def _tc_scorer(em2, grp, w1rep, b1r, w2, b2r):
  def body(em_ref, g_ref, w1_ref, b1_ref, w2_ref, b2_ref, o_ref):
    em = em_ref[...]
    g0 = g_ref[:, 0:1]
    g1 = g_ref[:, 1:2]
    lane = lax.broadcasted_iota(jnp.int32, (BLK, 2 * TW), 1)
    want = jnp.where(lane < TW, g0, g1)
    x = jnp.where(((lane >> 5) & 3) == want, em, 0.0)
    h = jnp.tanh(
        lax.dot_general(
            x, w1_ref[...], (((1,), (0,)), ((), ())),
            preferred_element_type=jnp.float32,
        )
        + b1_ref[...]
    )
    o_ref[...] = (
        lax.dot_general(
            h, w2_ref[...], (((1,), (0,)), ((), ())),
            preferred_element_type=jnp.float32,
        )
        + b2_ref[...]
    )

  return pl.pallas_call(
      body,
      grid=(B // BLK,),
      in_specs=[
          pl.BlockSpec((BLK, 2 * TW), lambda i: (i, 0)),
          pl.BlockSpec((BLK, 2), lambda i: (i, 0)),
          pl.BlockSpec((2 * TW, ATT), lambda i: (0, 0)),
          pl.BlockSpec((1, ATT), lambda i: (0, 0)),
          pl.BlockSpec((ATT, 1), lambda i: (0, 0)),
          pl.BlockSpec((1, 1), lambda i: (0, 0)),
      ],
      out_specs=pl.BlockSpec((BLK, 1), lambda i: (i, 0)),
      out_shape=jax.ShapeDtypeStruct((B, 1), jnp.float32),
  )(em2, grp, w1rep, b1r, w2, b2r)


def kernel(inds, mask, table, W1, b1, W2, b2):
  eye = jnp.eye(D, dtype=jnp.float32)
  e4 = jnp.concatenate(
      [jnp.pad(eye, ((0, 0), (32 * v, 96 - 32 * v))) for v in range(4)],
      axis=1).astype(jnp.bfloat16)
  table128 = _tc_pack(table.T, e4)
  rows = ((inds >> SH) << (SH - 2)) | (inds & (GW - 1))
  idx2d = rows.reshape(NW * NCH, CH)
  grp = (inds >> (SH - 2)) & 3
  em2 = _sc_gather(table128, idx2d).reshape(B, 2 * TW)
  w1rep = jnp.tile(W1, (8, 1)) * 0.5
  return _tc_scorer(em2, grp, w1rep, b1.reshape(1, ATT), W2,
                    b2.reshape(1, 1))


# speedup vs baseline: 6.9788x; 1.0980x over previous
import functools

import jax
import jax.numpy as jnp
from jax import lax
from jax.experimental import pallas as pl
from jax.experimental.pallas import tpu as pltpu
from jax.experimental.pallas import tpu_sc as plsc

B = 16384
D = 32
B2 = 2 * B
V = 1000000
NW = 32
BPW = B2 // NW
CH = 128
NCH = BPW // CH
TW = 128
BLKC = 32768
GW = BLKC // 4
SH = BLKC.bit_length() - 1
NGP = (V + BLKC - 1) // BLKC
TR = NGP * GW

ATT = 64
BLK = 1024


def _tc_pack(tableT, e4):
  def body(x_ref, e_ref, o_ref):
    x = x_ref[...].astype(jnp.bfloat16)
    xs = jnp.concatenate([x[:, GW * v:GW * (v + 1)] for v in range(4)],
                         axis=0)
    o_ref[...] = lax.dot_general(
        xs, e_ref[...], (((0,), (0,)), ((), ())),
        preferred_element_type=jnp.float32)

  return pl.pallas_call(
      body,
      grid=(NGP,),
      in_specs=[
          pl.BlockSpec((D, BLKC), lambda g: (0, g)),
          pl.BlockSpec((TW, TW), lambda g: (0, 0)),
      ],
      out_specs=pl.BlockSpec((GW, TW), lambda g: (g, 0)),
      out_shape=jax.ShapeDtypeStruct((TR, TW), jnp.float32),
  )(tableT, e4)


def _sc_gather(table128, idx2d):
  mesh = plsc.VectorSubcoreMesh(core_axis_name="c", subcore_axis_name="s")

  @functools.partial(
      pl.kernel,
      mesh=mesh,
      out_type=jax.ShapeDtypeStruct((B2, TW), jnp.float32),
      scratch_types=[
          pltpu.VMEM((NCH, CH), jnp.int32),
          pltpu.VMEM((CH, TW), jnp.float32),
          pltpu.VMEM((CH, TW), jnp.float32),
          pltpu.SemaphoreType.DMA,
          pltpu.SemaphoreType.DMA,
          pltpu.SemaphoreType.DMA,
          pltpu.SemaphoreType.DMA,
      ],
  )
  def k(table_hbm, idx_hbm, out_hbm, idx_v, b0, b1, gs0, gs1, ws0, ws1):
    wid = lax.axis_index("s") * 2 + lax.axis_index("c")
    pltpu.sync_copy(idx_hbm.at[pl.ds(wid * NCH, NCH)], idx_v)
    bufs = (b0, b1)
    gsem = (gs0, gs1)
    wsem = (ws0, ws1)
    gc = [None, None]
    wc = [None, None]
    for j in range(NCH):
      p = j % 2
      if wc[p] is not None:
        wc[p].wait()
      gc[p] = pltpu.async_copy(table_hbm.at[idx_v.at[j]], bufs[p], gsem[p])
      if j >= 1:
        q = (j - 1) % 2
        gc[q].wait()
        wc[q] = pltpu.async_copy(
            bufs[q], out_hbm.at[pl.ds((wid * NCH + j - 1) * CH, CH)], wsem[q])
    q = (NCH - 1) % 2
    gc[q].wait()
    wc[q] = pltpu.async_copy(
        bufs[q], out_hbm.at[pl.ds((wid * NCH + NCH - 1) * CH, CH)], wsem[q])
    wc[0].wait()
    wc[1].wait()

  return k(table128, idx2d)


def _tc_scorer(em2, grp, w1rep, b1r, w2, b2r):
  def body(em_ref, g_ref, w1_ref, b1_ref, w2_ref, b2_ref, o_ref):
    em = em_ref[...]
    g0 = g_ref[:, 0:1]
    g1 = g_ref[:, 1:2]
    lane = lax.broadcasted_iota(jnp.int32, (BLK, 2 * TW), 1)
    want = jnp.where(lane < TW, g0, g1)
    x = jnp.where(((lane >> 5) & 3) == want, em, 0.0)
    h = jnp.tanh(
        lax.dot_general(
            x, w1_ref[...], (((1,), (0,)), ((), ())),
            preferred_element_type=jnp.float32,
        )
        + b1_ref[...]
    )
    o_ref[...] = (
        lax.dot_general(
            h, w2_ref[...], (((1,), (0,)), ((), ())),
            preferred_element_type=jnp.float32,
        )
        + b2_ref[...]
    )

  return pl.pallas_call(
      body,
      grid=(B // BLK,),
      in_specs=[
          pl.BlockSpec((BLK, 2 * TW), lambda i: (i, 0)),
          pl.BlockSpec((BLK, 2), lambda i: (i, 0)),
          pl.BlockSpec((2 * TW, ATT), lambda i: (0, 0)),
          pl.BlockSpec((1, ATT), lambda i: (0, 0)),
          pl.BlockSpec((ATT, 1), lambda i: (0, 0)),
          pl.BlockSpec((1, 1), lambda i: (0, 0)),
      ],
      out_specs=pl.BlockSpec((BLK, 1), lambda i: (i, 0)),
      out_shape=jax.ShapeDtypeStruct((B, 1), jnp.float32),
  )(em2, grp, w1rep, b1r, w2, b2r)


def kernel(inds, mask, table, W1, b1, W2, b2):
  e4 = jnp.eye(TW, dtype=jnp.bfloat16)
  table128 = _tc_pack(table.T, e4)
  rows = ((inds >> SH) << (SH - 2)) | (inds & (GW - 1))
  idx2d = rows.reshape(NW * NCH, CH)
  grp = (inds >> (SH - 2)) & 3
  em2 = _sc_gather(table128, idx2d).reshape(B, 2 * TW)
  w1rep = jnp.tile(W1, (8, 1)) * 0.5
  return _tc_scorer(em2, grp, w1rep, b1.reshape(1, ATT), W2,
                    b2.reshape(1, 1))


# speedup vs baseline: 7.0023x; 1.0034x over previous
import functools

import jax
import jax.numpy as jnp
from jax import lax
from jax.experimental import pallas as pl
from jax.experimental.pallas import tpu as pltpu
from jax.experimental.pallas import tpu_sc as plsc

B = 16384
D = 32
B2 = 2 * B
V = 1000000
NW = 32
BPW = B2 // NW
CH = 128
NCH = BPW // CH
TW = 128
BLKC = 65536
GW = BLKC // 4
SH = BLKC.bit_length() - 1
NGP = (V + BLKC - 1) // BLKC
TR = NGP * GW

ATT = 64
BLK = 1024


def _tc_pack(tableT, e4):
  def body(x_ref, e_ref, o_ref):
    x = x_ref[...].astype(jnp.bfloat16)
    xs = jnp.concatenate([x[:, GW * v:GW * (v + 1)] for v in range(4)],
                         axis=0)
    o_ref[...] = lax.dot_general(
        xs, e_ref[...], (((0,), (0,)), ((), ())),
        preferred_element_type=jnp.float32)

  return pl.pallas_call(
      body,
      grid=(NGP,),
      in_specs=[
          pl.BlockSpec((D, BLKC), lambda g: (0, g)),
          pl.BlockSpec((TW, TW), lambda g: (0, 0)),
      ],
      out_specs=pl.BlockSpec((GW, TW), lambda g: (g, 0)),
      out_shape=jax.ShapeDtypeStruct((TR, TW), jnp.float32),
  )(tableT, e4)


def _sc_gather(table128, idx2d):
  mesh = plsc.VectorSubcoreMesh(core_axis_name="c", subcore_axis_name="s")

  @functools.partial(
      pl.kernel,
      mesh=mesh,
      out_type=jax.ShapeDtypeStruct((B2, TW), jnp.float32),
      scratch_types=[
          pltpu.VMEM((NCH, CH), jnp.int32),
          pltpu.VMEM((CH, TW), jnp.float32),
          pltpu.VMEM((CH, TW), jnp.float32),
          pltpu.SemaphoreType.DMA,
          pltpu.SemaphoreType.DMA,
          pltpu.SemaphoreType.DMA,
          pltpu.SemaphoreType.DMA,
      ],
  )
  def k(table_hbm, idx_hbm, out_hbm, idx_v, b0, b1, gs0, gs1, ws0, ws1):
    wid = lax.axis_index("s") * 2 + lax.axis_index("c")
    pltpu.sync_copy(idx_hbm.at[pl.ds(wid * NCH, NCH)], idx_v)
    bufs = (b0, b1)
    gsem = (gs0, gs1)
    wsem = (ws0, ws1)
    gc = [None, None]
    wc = [None, None]
    for j in range(NCH):
      p = j % 2
      if wc[p] is not None:
        wc[p].wait()
      gc[p] = pltpu.async_copy(table_hbm.at[idx_v.at[j]], bufs[p], gsem[p])
      if j >= 1:
        q = (j - 1) % 2
        gc[q].wait()
        wc[q] = pltpu.async_copy(
            bufs[q], out_hbm.at[pl.ds((wid * NCH + j - 1) * CH, CH)], wsem[q])
    q = (NCH - 1) % 2
    gc[q].wait()
    wc[q] = pltpu.async_copy(
        bufs[q], out_hbm.at[pl.ds((wid * NCH + NCH - 1) * CH, CH)], wsem[q])
    wc[0].wait()
    wc[1].wait()

  return k(table128, idx2d)


def _tc_scorer(em2, grp, w1rep, b1r, w2, b2r):
  def body(em_ref, g_ref, w1_ref, b1_ref, w2_ref, b2_ref, o_ref):
    em = em_ref[...].astype(jnp.float32)
    g0 = g_ref[:, 0:1]
    g1 = g_ref[:, 1:2]
    lane = lax.broadcasted_iota(jnp.int32, (BLK, 2 * TW), 1)
    want = jnp.where(lane < TW, g0, g1)
    x = jnp.where(((lane >> 5) & 3) == want, em, 0.0)
    h = jnp.tanh(
        lax.dot_general(
            x, w1_ref[...], (((1,), (0,)), ((), ())),
            preferred_element_type=jnp.float32,
        )
        + b1_ref[...]
    )
    o_ref[...] = (
        lax.dot_general(
            h, w2_ref[...], (((1,), (0,)), ((), ())),
            preferred_element_type=jnp.float32,
        )
        + b2_ref[...]
    )

  return pl.pallas_call(
      body,
      grid=(B // BLK,),
      in_specs=[
          pl.BlockSpec((BLK, 2 * TW), lambda i: (i, 0)),
          pl.BlockSpec((BLK, 2), lambda i: (i, 0)),
          pl.BlockSpec((2 * TW, ATT), lambda i: (0, 0)),
          pl.BlockSpec((1, ATT), lambda i: (0, 0)),
          pl.BlockSpec((ATT, 1), lambda i: (0, 0)),
          pl.BlockSpec((1, 1), lambda i: (0, 0)),
      ],
      out_specs=pl.BlockSpec((BLK, 1), lambda i: (i, 0)),
      out_shape=jax.ShapeDtypeStruct((B, 1), jnp.float32),
  )(em2, grp, w1rep, b1r, w2, b2r)


def kernel(inds, mask, table, W1, b1, W2, b2):
  e4 = jnp.eye(TW, dtype=jnp.bfloat16)
  table128 = _tc_pack(table.T, e4)
  rows = ((inds >> SH) << (SH - 2)) | (inds & (GW - 1))
  idx2d = rows.reshape(NW * NCH, CH)
  grp = (inds >> (SH - 2)) & 3
  em2 = _sc_gather(table128, idx2d).reshape(B, 2 * TW)
  w1rep = jnp.tile(W1, (8, 1)) * 0.5
  return _tc_scorer(em2, grp, w1rep, b1.reshape(1, ATT), W2,
                    b2.reshape(1, 1))


# speedup vs baseline: 8.3769x; 1.1963x over previous
import functools

import jax
import jax.numpy as jnp
from jax import lax
from jax.experimental import pallas as pl
from jax.experimental.pallas import tpu as pltpu
from jax.experimental.pallas import tpu_sc as plsc

B = 16384
D = 32
B2 = 2 * B
V = 1000000
NW = 32
BPW = B2 // NW
CH = 128
NCH = BPW // CH
TW = 128
BLKC = 65536
GW = BLKC // 4
SH = BLKC.bit_length() - 1
NGP = (V + BLKC - 1) // BLKC
TR = NGP * GW

ATT = 64
BLK = 1024


def _tc_pack(tableT, e4):
  def body(x_ref, e_ref, o_ref):
    x = x_ref[...].astype(jnp.bfloat16)
    xs = jnp.concatenate([x[:, GW * v:GW * (v + 1)] for v in range(4)],
                         axis=0)
    o_ref[...] = lax.dot_general(
        xs, e_ref[...], (((0,), (0,)), ((), ())),
        preferred_element_type=jnp.float32)

  return pl.pallas_call(
      body,
      grid=(NGP,),
      in_specs=[
          pl.BlockSpec((D, BLKC), lambda g: (0, g)),
          pl.BlockSpec((TW, TW), lambda g: (0, 0)),
      ],
      out_specs=pl.BlockSpec((GW, TW), lambda g: (g, 0)),
      out_shape=jax.ShapeDtypeStruct((TR, TW), jnp.float32),
  )(tableT, e4)


def _sc_gather(table128, idx2d):
  mesh = plsc.VectorSubcoreMesh(core_axis_name="c", subcore_axis_name="s")
  HC = CH // 2

  @functools.partial(
      pl.kernel,
      mesh=mesh,
      out_type=jax.ShapeDtypeStruct((B, 2 * TW), jnp.float32),
      scratch_types=[
          pltpu.VMEM((NCH, CH), jnp.int32),
          pltpu.VMEM((HC, TW), jnp.float32),
          pltpu.VMEM((HC, TW), jnp.float32),
          pltpu.VMEM((HC, TW), jnp.float32),
          pltpu.VMEM((HC, TW), jnp.float32),
          pltpu.SemaphoreType.DMA,
          pltpu.SemaphoreType.DMA,
          pltpu.SemaphoreType.DMA,
          pltpu.SemaphoreType.DMA,
          pltpu.SemaphoreType.DMA,
          pltpu.SemaphoreType.DMA,
          pltpu.SemaphoreType.DMA,
          pltpu.SemaphoreType.DMA,
      ],
  )
  def k(table_hbm, idx_hbm, out_hbm, idx_v, bA0, bA1, bB0, bB1,
        gA0, gA1, gB0, gB1, wA0, wA1, wB0, wB1):
    wid = lax.axis_index("s") * 2 + lax.axis_index("c")
    pltpu.sync_copy(idx_hbm.at[pl.ds(wid * NCH, NCH)], idx_v)
    bufsA = (bA0, bA1)
    bufsB = (bB0, bB1)
    gsemA = (gA0, gA1)
    gsemB = (gB0, gB1)
    wsemA = (wA0, wA1)
    wsemB = (wB0, wB1)
    gcA = [None, None]
    gcB = [None, None]
    wcA = [None, None]
    wcB = [None, None]

    def issue_write(q, j):
      gcA[q].wait()
      gcB[q].wait()
      base = (wid * NCH + j) * HC
      wcA[q] = pltpu.async_copy(
          bufsA[q], out_hbm.at[pl.ds(base, HC), pl.ds(0, TW)], wsemA[q])
      wcB[q] = pltpu.async_copy(
          bufsB[q], out_hbm.at[pl.ds(base, HC), pl.ds(TW, TW)], wsemB[q])

    for j in range(NCH):
      p = j % 2
      if wcA[p] is not None:
        wcA[p].wait()
        wcB[p].wait()
      gcA[p] = pltpu.async_copy(
          table_hbm.at[idx_v.at[j, pl.ds(0, HC)]], bufsA[p], gsemA[p])
      gcB[p] = pltpu.async_copy(
          table_hbm.at[idx_v.at[j, pl.ds(HC, HC)]], bufsB[p], gsemB[p])
      if j >= 1:
        issue_write((j - 1) % 2, j - 1)
    issue_write((NCH - 1) % 2, NCH - 1)
    wcA[0].wait()
    wcA[1].wait()
    wcB[0].wait()
    wcB[1].wait()

  return k(table128, idx2d)


def _tc_scorer(em2, grp, w1rep, b1r, w2, b2r):
  def body(em_ref, g_ref, w1_ref, b1_ref, w2_ref, b2_ref, o_ref):
    em = em_ref[...].astype(jnp.float32)
    g0 = g_ref[:, 0:1]
    g1 = g_ref[:, 1:2]
    lane = lax.broadcasted_iota(jnp.int32, (BLK, 2 * TW), 1)
    want = jnp.where(lane < TW, g0, g1)
    x = jnp.where(((lane >> 5) & 3) == want, em, 0.0)
    h = jnp.tanh(
        lax.dot_general(
            x, w1_ref[...], (((1,), (0,)), ((), ())),
            preferred_element_type=jnp.float32,
        )
        + b1_ref[...]
    )
    o_ref[...] = (
        lax.dot_general(
            h, w2_ref[...], (((1,), (0,)), ((), ())),
            preferred_element_type=jnp.float32,
        )
        + b2_ref[...]
    )

  return pl.pallas_call(
      body,
      grid=(B // BLK,),
      in_specs=[
          pl.BlockSpec((BLK, 2 * TW), lambda i: (i, 0)),
          pl.BlockSpec((BLK, 2), lambda i: (i, 0)),
          pl.BlockSpec((2 * TW, ATT), lambda i: (0, 0)),
          pl.BlockSpec((1, ATT), lambda i: (0, 0)),
          pl.BlockSpec((ATT, 1), lambda i: (0, 0)),
          pl.BlockSpec((1, 1), lambda i: (0, 0)),
      ],
      out_specs=pl.BlockSpec((BLK, 1), lambda i: (i, 0)),
      out_shape=jax.ShapeDtypeStruct((B, 1), jnp.float32),
  )(em2, grp, w1rep, b1r, w2, b2r)


def kernel(inds, mask, table, W1, b1, W2, b2):
  e4 = jnp.eye(TW, dtype=jnp.bfloat16)
  table128 = _tc_pack(table.T, e4)
  rows = ((inds >> SH) << (SH - 2)) | (inds & (GW - 1))
  idx2d = jnp.concatenate(
      [rows[:, 0].reshape(NW * NCH, CH // 2),
       rows[:, 1].reshape(NW * NCH, CH // 2)], axis=1)
  grp = (inds >> (SH - 2)) & 3
  em2 = _sc_gather(table128, idx2d)
  w1rep = jnp.tile(W1, (8, 1)) * 0.5
  return _tc_scorer(em2, grp, w1rep, b1.reshape(1, ATT), W2,
                    b2.reshape(1, 1))
